# baseline (speedup 1.0000x reference)
import functools

import jax
import jax.numpy as jnp
from jax import lax
from jax.experimental import pallas as pl
from jax.experimental.pallas import tpu as pltpu
from jax.experimental.pallas import tpu_sc as plsc

VOCAB = 1000000
EMBED = 32
BATCH = 16384

SLAB = 128
NSLAB = (VOCAB + SLAB - 1) // SLAB
LAST_TILE = (VOCAB // SLAB) * SLAB
LAST_W = VOCAB - LAST_TILE

_info = plsc.get_sparse_core_info()
_NC, _NS, _L = _info.num_cores, _info.num_subcores, _info.num_lanes
_NW = _NC * _NS
SLABS_PER_W = (NSLAB + _NW - 1) // _NW
_NWIN = BATCH // _L

CAP = 512
NCHUNK = CAP // SLAB


def _make_kernel():
  mesh = plsc.VectorSubcoreMesh(core_axis_name="c", subcore_axis_name="s")

  @functools.partial(
      pl.kernel,
      mesh=mesh,
      out_type=jax.ShapeDtypeStruct((BATCH + SLAB, SLAB), jnp.float32),
      compiler_params=pltpu.CompilerParams(needs_layout_passes=False),
      scratch_types=[
          pltpu.VMEM((BATCH + _L,), jnp.int32),
          pltpu.VMEM((BATCH + _L,), jnp.int32),
          pltpu.VMEM((BATCH + _L,), jnp.int32),
          pltpu.VMEM((2, EMBED, SLAB), jnp.float32),
          pltpu.VMEM((EMBED, LAST_W), jnp.float32),
          pltpu.VMEM((CAP, SLAB), jnp.float32),
          pltpu.VMEM((NCHUNK, SLAB), jnp.int32),
          pltpu.SemaphoreType.DMA((2,)),
          pltpu.SemaphoreType.DMA,
      ],
  )
  def gather_kernel(tT_hbm, idx_hbm, out_hbm, idx_v, pos_v, spos_v, slabs_v,
                    last_v, rows_v, rpos_v, sems, osem):
    wid = lax.axis_index("s") * _NC + lax.axis_index("c")
    slab_lo = wid * SLABS_PER_W
    my_nslab = lax.min(jnp.int32(NSLAB) - slab_lo, jnp.int32(SLABS_PER_W))

    pltpu.sync_copy(idx_hbm, idx_v.at[pl.ds(0, BATCH)])
    pltpu.sync_copy(tT_hbm.at[:, pl.ds(LAST_TILE, LAST_W)], last_v)

    kvec0 = lax.iota(jnp.int32, _L)
    kvec1 = kvec0 + _L
    lane0 = kvec0 == 0

    def reset_rpos():
      for cch in range(NCHUNK):
        for w in range(SLAB // _L):
          sent = jnp.full((_L,), BATCH + cch * (SLAB // NCHUNK) + w % (SLAB // NCHUNK // 2), jnp.int32) + kvec0
          rpos_v[cch, pl.ds(w * _L, _L)] = sent

    reset_rpos()

    def scan_body(w, cnt):
      v = idx_v[pl.ds(w * _L, _L)]
      slab = lax.shift_right_logical(v, 7)
      m = (slab >= slab_lo) & (slab < slab_lo + my_nslab)
      plsc.store_compressed(pos_v.at[pl.ds(cnt, _L)], w * _L + kvec0, mask=m)
      return cnt + plsc.all_reduce_population_count(m)[0]

    my_cnt = lax.fori_loop(0, _NWIN, scan_body, jnp.int32(0))

    def slab_base(s):
      return pl.multiple_of(
          lax.min(s * SLAB, jnp.int32(LAST_TILE - SLAB)), 128)

    def issue(s, buf):
      @pl.when(s < my_nslab)
      def _():
        pltpu.async_copy(
            tT_hbm.at[:, pl.ds(slab_base(slab_lo + s), SLAB)],
            slabs_v.at[buf], sems.at[buf])

    issue(jnp.int32(0), 0)
    issue(jnp.int32(1), 1)

    def flush(outcnt):
      copies = []
      for cch in range(NCHUNK):
        copies.append(pltpu.async_copy(
            rows_v.at[pl.ds(cch * SLAB, SLAB)],
            out_hbm.at[rpos_v.at[cch]],
            osem))
      for c in copies:
        c.wait()
      reset_rpos()
      return jnp.int32(0)

    def slab_step(s, outcnt):
      for j in range(2):
        sj = s * 2 + j
        glob = slab_lo + sj

        @pl.when(sj < my_nslab)
        def _():
          pltpu.make_async_copy(
              tT_hbm.at[:, pl.ds(0, SLAB)], slabs_v.at[j], sems.at[j]).wait()

        def match_body(w, cnt2):
          p = pos_v[pl.ds(w * _L, _L)]
          live = (w * _L + kvec0) < my_cnt
          p_safe = lax.clamp(jnp.int32(0), p, jnp.int32(BATCH - 1))
          r = plsc.load_gather(idx_v, [p_safe])
          m = live & (lax.shift_right_logical(r, 7) == glob)
          plsc.store_compressed(spos_v.at[pl.ds(cnt2, _L)], p, mask=m)
          return cnt2 + plsc.all_reduce_population_count(m)[0]

        nwin = lax.div(my_cnt + (_L - 1), jnp.int32(_L))
        cnt2 = lax.fori_loop(0, nwin, match_body, jnp.int32(0))

        def item_body(t, oc):
          p = spos_v[pl.ds(t, _L)][0]
          r = idx_v[pl.ds(p, _L)][0]
          iv = jnp.full((_L,), 0, jnp.int32) + oc

          @pl.when(r < LAST_TILE)
          def _():
            colv = jnp.full((_L,), 0, jnp.int32) + (r - slab_base(glob))
            v0 = plsc.load_gather(slabs_v.at[j], [kvec0, colv])
            v1 = plsc.load_gather(slabs_v.at[j], [kvec1, colv])
            plsc.store_scatter(rows_v, [iv, kvec0], v0)
            plsc.store_scatter(rows_v, [iv, kvec1], v1)

          @pl.when(r >= LAST_TILE)
          def _():
            cv = jnp.full((_L,), 0, jnp.int32) + (r - LAST_TILE)
            v0 = plsc.load_gather(last_v, [kvec0, cv])
            v1 = plsc.load_gather(last_v, [kvec1, cv])
            plsc.store_scatter(rows_v, [iv, kvec0], v0)
            plsc.store_scatter(rows_v, [iv, kvec1], v1)

          pv = jnp.full((_L,), 0, jnp.int32) + p
          plsc.store_scatter(
              rpos_v, [lax.shift_right_logical(iv, 7),
                       lax.bitwise_and(iv, SLAB - 1)], pv, mask=lane0)
          oc = oc + 1
          return lax.cond(oc >= CAP, flush, lambda x: x, oc)

        outcnt = lax.fori_loop(0, cnt2, item_body, outcnt)
        issue(sj + 2, j)
      return outcnt

    outcnt = lax.fori_loop(0, (SLABS_PER_W + 1) // 2, slab_step, jnp.int32(0))
    flush(outcnt)

  return gather_kernel


_gather = _make_kernel()


@jax.jit
def kernel(data, table):
  padded = _gather(table.T, data.astype(jnp.int32))
  return padded[:BATCH, :EMBED]


# speedup vs baseline: 1.1499x; 1.1499x over previous
import functools

import jax
import jax.numpy as jnp
from jax import lax
from jax.experimental import pallas as pl
from jax.experimental.pallas import tpu as pltpu
from jax.experimental.pallas import tpu_sc as plsc

VOCAB = 1000000
EMBED = 32
BATCH = 16384

SLAB = 128
NSLAB = (VOCAB + SLAB - 1) // SLAB
LAST_TILE = (VOCAB // SLAB) * SLAB
LAST_W = VOCAB - LAST_TILE

_info = plsc.get_sparse_core_info()
_NC, _NS, _L = _info.num_cores, _info.num_subcores, _info.num_lanes
_NW = _NC * _NS
SLABS_PER_W = (NSLAB + _NW - 1) // _NW
_NWIN = BATCH // _L

CAP = 512
NCHUNK = CAP // SLAB


def _make_kernel():
  mesh = plsc.VectorSubcoreMesh(core_axis_name="c", subcore_axis_name="s")

  @functools.partial(
      pl.kernel,
      mesh=mesh,
      out_type=jax.ShapeDtypeStruct((BATCH + SLAB, SLAB), jnp.float32),
      compiler_params=pltpu.CompilerParams(needs_layout_passes=False),
      scratch_types=[
          pltpu.VMEM((BATCH + _L,), jnp.int32),
          pltpu.VMEM((BATCH + _L,), jnp.int32),
          pltpu.VMEM((BATCH + _L,), jnp.int32),
          pltpu.VMEM((SLABS_PER_W + _L,), jnp.int32),
          pltpu.VMEM((SLABS_PER_W + _L,), jnp.int32),
          pltpu.VMEM((2, EMBED, SLAB), jnp.float32),
          pltpu.VMEM((EMBED, LAST_W), jnp.float32),
          pltpu.VMEM((CAP, SLAB), jnp.float32),
          pltpu.VMEM((NCHUNK, SLAB), jnp.int32),
          pltpu.SemaphoreType.DMA((2,)),
          pltpu.SemaphoreType.DMA,
      ],
  )
  def gather_kernel(tT_hbm, idx_hbm, out_hbm, idx_v, pos_v, ord_v, st0_v,
                    cur_v, slabs_v, last_v, rows_v, rpos_v, sems, osem):
    wid = lax.axis_index("s") * _NC + lax.axis_index("c")
    slab_lo = wid * SLABS_PER_W
    my_nslab = lax.min(jnp.int32(NSLAB) - slab_lo, jnp.int32(SLABS_PER_W))

    pltpu.sync_copy(idx_hbm, idx_v.at[pl.ds(0, BATCH)])
    pltpu.sync_copy(tT_hbm.at[:, pl.ds(LAST_TILE, LAST_W)], last_v)

    kvec0 = lax.iota(jnp.int32, _L)
    kvec1 = kvec0 + _L
    lane0 = kvec0 == 0

    def reset_rpos():
      for cch in range(NCHUNK):
        for w in range(SLAB // _L):
          sent = jnp.full((_L,), BATCH + cch * (SLAB // NCHUNK) + w % (SLAB // NCHUNK // 2), jnp.int32) + kvec0
          rpos_v[cch, pl.ds(w * _L, _L)] = sent

    reset_rpos()

    def scan_body(w, cnt):
      v = idx_v[pl.ds(w * _L, _L)]
      slab = lax.shift_right_logical(v, 7)
      m = (slab >= slab_lo) & (slab < slab_lo + my_nslab)
      plsc.store_compressed(pos_v.at[pl.ds(cnt, _L)], w * _L + kvec0, mask=m)
      return cnt + plsc.all_reduce_population_count(m)[0]

    my_cnt = lax.fori_loop(0, _NWIN, scan_body, jnp.int32(0))

    zero = jnp.full((_L,), 0, jnp.int32)
    for w in range((SLABS_PER_W + _L) // _L):
      cur_v[pl.ds(w * _L, _L)] = zero

    def count_body(t, _):
      p = pos_v[pl.ds(t, _L)][0]
      r = idx_v[pl.ds(p, _L)][0]
      lv = jnp.full((_L,), 0, jnp.int32) + (lax.shift_right_logical(r, 7)
                                            - slab_lo)
      c = plsc.load_gather(cur_v, [lv])
      plsc.store_scatter(cur_v, [lv], c + 1, mask=lane0)
      return ()

    lax.fori_loop(0, my_cnt, count_body, ())

    def cumsum_body(w, tot):
      c = cur_v[pl.ds(w * _L, _L)]
      cs = plsc.cumsum(c)
      st0_v[pl.ds(w * _L, _L)] = cs - c + tot
      cur_v[pl.ds(w * _L, _L)] = cs - c + tot
      return tot + lax.convert_element_type(cs[_L - 1], jnp.int32)

    lax.fori_loop(0, (SLABS_PER_W + _L) // _L, cumsum_body, jnp.int32(0))

    def place_body(t, _):
      p = pos_v[pl.ds(t, _L)][0]
      r = idx_v[pl.ds(p, _L)][0]
      lv = jnp.full((_L,), 0, jnp.int32) + (lax.shift_right_logical(r, 7)
                                            - slab_lo)
      o = plsc.load_gather(cur_v, [lv])
      plsc.store_scatter(cur_v, [lv], o + 1, mask=lane0)
      plsc.store_scatter(ord_v, [o], jnp.full((_L,), 0, jnp.int32) + p,
                         mask=lane0)
      return ()

    lax.fori_loop(0, my_cnt, place_body, ())

    def slab_base(s):
      return pl.multiple_of(
          lax.min(s * SLAB, jnp.int32(LAST_TILE - SLAB)), 128)

    def issue(s, buf):
      @pl.when(s < my_nslab)
      def _():
        pltpu.async_copy(
            tT_hbm.at[:, pl.ds(slab_base(slab_lo + s), SLAB)],
            slabs_v.at[buf], sems.at[buf])

    issue(jnp.int32(0), 0)
    issue(jnp.int32(1), 1)

    def flush(outcnt):
      copies = []
      for cch in range(NCHUNK):
        copies.append(pltpu.async_copy(
            rows_v.at[pl.ds(cch * SLAB, SLAB)],
            out_hbm.at[rpos_v.at[cch]],
            osem))
      for c in copies:
        c.wait()
      reset_rpos()
      return jnp.int32(0)

    def slab_step(s, outcnt):
      for j in range(2):
        sj = s * 2 + j
        glob = slab_lo + sj

        @pl.when(sj < my_nslab)
        def _():
          pltpu.make_async_copy(
              tT_hbm.at[:, pl.ds(0, SLAB)], slabs_v.at[j], sems.at[j]).wait()

        lo2 = st0_v[pl.ds(lax.min(sj, my_nslab - 1), _L)][0]
        hi2 = cur_v[pl.ds(lax.min(sj, my_nslab - 1), _L)][0]
        hi2 = lax.select(sj < my_nslab, hi2, lo2)

        def item_body(t, oc):
          p = ord_v[pl.ds(t, _L)][0]
          r = idx_v[pl.ds(p, _L)][0]
          iv = jnp.full((_L,), 0, jnp.int32) + oc

          @pl.when(r < LAST_TILE)
          def _():
            colv = jnp.full((_L,), 0, jnp.int32) + (r - slab_base(glob))
            v0 = plsc.load_gather(slabs_v.at[j], [kvec0, colv])
            v1 = plsc.load_gather(slabs_v.at[j], [kvec1, colv])
            plsc.store_scatter(rows_v, [iv, kvec0], v0)
            plsc.store_scatter(rows_v, [iv, kvec1], v1)

          @pl.when(r >= LAST_TILE)
          def _():
            cv = jnp.full((_L,), 0, jnp.int32) + (r - LAST_TILE)
            v0 = plsc.load_gather(last_v, [kvec0, cv])
            v1 = plsc.load_gather(last_v, [kvec1, cv])
            plsc.store_scatter(rows_v, [iv, kvec0], v0)
            plsc.store_scatter(rows_v, [iv, kvec1], v1)

          pv = jnp.full((_L,), 0, jnp.int32) + p
          plsc.store_scatter(
              rpos_v, [lax.shift_right_logical(iv, 7),
                       lax.bitwise_and(iv, SLAB - 1)], pv, mask=lane0)
          oc = oc + 1
          return lax.cond(oc >= CAP, flush, lambda x: x, oc)

        outcnt = lax.fori_loop(lo2, hi2, item_body, outcnt)
        issue(sj + 2, j)
      return outcnt

    outcnt = lax.fori_loop(0, (SLABS_PER_W + 1) // 2, slab_step, jnp.int32(0))
    flush(outcnt)

  return gather_kernel


_gather = _make_kernel()


@jax.jit
def kernel(data, table):
  padded = _gather(table.T, data.astype(jnp.int32))
  return padded[:BATCH, :EMBED]


# speedup vs baseline: 1.9821x; 1.7238x over previous
import functools

import jax
import jax.numpy as jnp
from jax import lax
from jax.experimental import pallas as pl
from jax.experimental.pallas import tpu as pltpu
from jax.experimental.pallas import tpu_sc as plsc

VOCAB = 1000000
EMBED = 32
BATCH = 16384

SLAB = 128
LAST_TILE = (VOCAB // SLAB) * SLAB
LAST_W = VOCAB - LAST_TILE
MAX_RC = LAST_TILE - SLAB
RING = 8

_info = plsc.get_sparse_core_info()
_NC, _NS, _L = _info.num_cores, _info.num_subcores, _info.num_lanes
_NW = _NC * _NS
_B_PER_W = BATCH // _NW


def _make_kernel():
  mesh = plsc.VectorSubcoreMesh(core_axis_name="c", subcore_axis_name="s")

  @functools.partial(
      pl.kernel,
      mesh=mesh,
      out_type=jax.ShapeDtypeStruct((EMBED, BATCH), jnp.float32),
      compiler_params=pltpu.CompilerParams(needs_layout_passes=False),
      scratch_types=[
          pltpu.VMEM((_B_PER_W + _L,), jnp.int32),
          pltpu.VMEM((RING, EMBED, SLAB), jnp.float32),
          pltpu.VMEM((EMBED, LAST_W), jnp.float32),
          pltpu.VMEM((EMBED, _B_PER_W), jnp.float32),
          pltpu.SemaphoreType.DMA((RING,)),
      ],
  )
  def gather_kernel(tT_hbm, idx_hbm, out_hbm, idx_v, slabs_v, last_v,
                    cols_v, sems):
    wid = lax.axis_index("s") * _NC + lax.axis_index("c")
    base = pl.multiple_of(wid * _B_PER_W, 128)
    pltpu.sync_copy(idx_hbm.at[pl.ds(base, _B_PER_W)],
                    idx_v.at[pl.ds(0, _B_PER_W)])
    pltpu.sync_copy(tT_hbm.at[:, pl.ds(LAST_TILE, LAST_W)], last_v)

    kvec0 = lax.iota(jnp.int32, _L)
    kvec1 = kvec0 + _L

    def read_idx(i):
      return idx_v[pl.ds(i, _L)][0]

    def rc_of(r):
      rc = lax.shift_left(lax.shift_right_logical(r, 7), 7)
      return pl.multiple_of(lax.min(rc, MAX_RC), 128)

    def issue(i, buf):
      r = read_idx(i)
      pltpu.async_copy(tT_hbm.at[:, pl.ds(rc_of(r), SLAB)],
                       slabs_v.at[buf], sems.at[buf])

    for j in range(RING):
      issue(j, j)

    def body(c, _):
      for j in range(RING):
        i = c * RING + j
        r = read_idx(i)
        pltpu.make_async_copy(tT_hbm.at[:, pl.ds(0, SLAB)],
                              slabs_v.at[j], sems.at[j]).wait()
        iv = jnp.full((_L,), 0, jnp.int32) + i

        @pl.when(r < LAST_TILE)
        def _():
          colv = jnp.full((_L,), 0, jnp.int32) + (r - rc_of(r))
          v0 = plsc.load_gather(slabs_v.at[j], [kvec0, colv])
          v1 = plsc.load_gather(slabs_v.at[j], [kvec1, colv])
          plsc.store_scatter(cols_v, [kvec0, iv], v0)
          plsc.store_scatter(cols_v, [kvec1, iv], v1)

        @pl.when(r >= LAST_TILE)
        def _():
          cv = jnp.full((_L,), 0, jnp.int32) + (r - LAST_TILE)
          v0 = plsc.load_gather(last_v, [kvec0, cv])
          v1 = plsc.load_gather(last_v, [kvec1, cv])
          plsc.store_scatter(cols_v, [kvec0, iv], v0)
          plsc.store_scatter(cols_v, [kvec1, iv], v1)

        nxt = i + RING

        @pl.when(nxt < _B_PER_W)
        def _():
          issue(nxt, j)
      return ()

    lax.fori_loop(0, _B_PER_W // RING, body, ())
    pltpu.sync_copy(cols_v, out_hbm.at[:, pl.ds(base, _B_PER_W)])

  return gather_kernel


_gather = _make_kernel()


@jax.jit
def kernel(data, table):
  outT = _gather(table.T, data.astype(jnp.int32))
  return outT.T
